# baseline (speedup 1.0000x reference)
import numpy as np
import jax
import jax.numpy as jnp
from jax import lax
from jax.experimental import pallas as pl
from jax.experimental.pallas import tpu as pltpu
from jax.experimental.pallas import tpu_sc as plsc

_PRIOR_BOXES = np.array([[1.3221, 1.73145], [3.19275, 4.00944], [5.05587, 8.09892],
                         [9.47112, 4.84053], [11.2364, 10.0071]], dtype=np.float32) / 13.0
_IMG_W = 416.0
_IMG_H = 416.0
_LAMBDA_OBJ = 5.0
_LAMBDA_NONOBJ = 0.5

_B, _C, _H, _W = 64, 125, 52, 52
_HW = _H * _W
_NV = _HW // 16
_K = 4
_NWORK = 32
_BPW = _B // _NWORK
_CHUNKS = [(c0, min(_K, _C - c0)) for c0 in range(0, _C, _K)]
_W0S = (0, 16, 32, 36)

_DX = np.float32(_IMG_W / _C)
_DY = np.float32(_IMG_H / _C)


def _grid_vecs():
    cell_x = np.tile(np.arange(_W, dtype=np.float32), _H)
    cell_y = np.repeat(np.arange(_H, dtype=np.float32), _W)
    return np.concatenate([_DX * cell_x, _DY * cell_y])


def _emit_plane_row(c, j, r, ib, ob, ms, gxs, gys):
    pos, anchor = c % 25, c // 25
    for i, w0 in enumerate(_W0S):
        sl = pl.ds(w0, 16)
        p = ib[j, r, sl]
        if pos == 0 or pos >= 5:
            ob[j, r, sl] = p * ms[i]
        elif pos == 1:
            t = (_DX * p).astype(jnp.int32).astype(jnp.float32)
            ob[j, r, sl] = (t + gxs[i]) * ms[i]
        elif pos == 2:
            t = (_DY * p).astype(jnp.int32).astype(jnp.float32)
            ob[j, r, sl] = (t + gys[i]) * ms[i]
        elif pos == 3:
            pw = float(_PRIOR_BOXES[anchor, 0])
            t = ((pw * p) * _IMG_W).astype(jnp.int32).astype(jnp.float32)
            ob[j, r, sl] = t * ms[i]
        else:
            ph = float(_PRIOR_BOXES[anchor, 1])
            t = ((ph * p) * _IMG_H).astype(jnp.int32).astype(jnp.float32)
            ob[j, r, sl] = t * ms[i]


def _sc_body(pred_hbm, y0_hbm, gxy_hbm, out_hbm,
             gx_v, gy_v, mask_v, in0, in1, out0, out1,
             sin0, sin1, sout0, sout1):
    wid = lax.axis_index("s") * 2 + lax.axis_index("c")
    pltpu.sync_copy(gxy_hbm.at[pl.ds(0, _HW)], gx_v)
    pltpu.sync_copy(gxy_hbm.at[pl.ds(_HW, _HW)], gy_v)
    in_bufs, in_sems = (in0, in1), (sin0, sin1)
    out_bufs, out_sems = (out0, out1), (sout0, sout1)
    nch = len(_CHUNKS)

    def batch_body(bi, _):
        b = wid * _BPW + bi
        ybase = pl.multiple_of(b * _HW, 8)

        pltpu.sync_copy(y0_hbm.at[pl.ds(ybase, _HW)], mask_v)

        def mask_body(v, _):
            sl = pl.ds(v * 16, 16)
            y = mask_v[sl]
            mask_v[sl] = _LAMBDA_OBJ * y + _LAMBDA_NONOBJ * jnp.negative(y + (-1.0))
            return 0

        lax.fori_loop(0, _NV, mask_body, 0)

        def in_cp(ch, buf, sem):
            c0, sz = _CHUNKS[ch]
            return pltpu.make_async_copy(
                pred_hbm.at[b, pl.ds(c0, sz)], buf.at[pl.ds(0, sz)], sem)

        def out_cp(ch, buf, sem):
            c0, sz = _CHUNKS[ch]
            return pltpu.make_async_copy(
                buf.at[pl.ds(0, sz)], out_hbm.at[b, pl.ds(c0, sz)], sem)

        in_cp(0, in_bufs[0], in_sems[0]).start()
        for ch in range(nch):
            cur = ch % 2
            c0, sz = _CHUNKS[ch]
            in_cp(ch, in_bufs[cur], in_sems[cur]).wait()
            if ch + 1 < nch:
                in_cp(ch + 1, in_bufs[1 - cur], in_sems[1 - cur]).start()
            if ch >= 2:
                out_cp(ch - 2, out_bufs[cur], out_sems[cur]).wait()
            ib, ob = in_bufs[cur], out_bufs[cur]
            rows = [c0 + j for j in range(sz)]
            need_gx = any(c % 25 == 1 for c in rows)
            need_gy = any(c % 25 == 2 for c in rows)

            def body(r, _, ib=ib, ob=ob, rows=rows,
                     need_gx=need_gx, need_gy=need_gy):
                ro = r * _W
                ms = [mask_v[pl.ds(ro + w0, 16)] for w0 in _W0S]
                gxs = [gx_v[pl.ds(ro + w0, 16)] for w0 in _W0S] if need_gx else None
                gys = [gy_v[pl.ds(ro + w0, 16)] for w0 in _W0S] if need_gy else None
                for j, c in enumerate(rows):
                    _emit_plane_row(c, j, r, ib, ob, ms, gxs, gys)
                return 0

            lax.fori_loop(0, _H, body, 0)
            out_cp(ch, out_bufs[cur], out_sems[cur]).start()
        out_cp(nch - 2, out_bufs[(nch - 2) % 2], out_sems[(nch - 2) % 2]).wait()
        out_cp(nch - 1, out_bufs[(nch - 1) % 2], out_sems[(nch - 1) % 2]).wait()
        return 0

    lax.fori_loop(0, _BPW, batch_body, 0)


def kernel(pred, y_hat):
    B, C, H, W = pred.shape
    HW = H * W
    gxy = _grid_vecs()

    y0 = y_hat[:, :, :, 0].reshape(B * HW)

    mesh = plsc.VectorSubcoreMesh(core_axis_name="c", subcore_axis_name="s")
    sc = pl.kernel(
        _sc_body,
        mesh=mesh,
        out_type=jax.ShapeDtypeStruct((B, C, H, W), jnp.float32),
        scratch_types=[
            pltpu.VMEM((_HW,), jnp.float32),
            pltpu.VMEM((_HW,), jnp.float32),
            pltpu.VMEM((_HW,), jnp.float32),
            pltpu.VMEM((_K, _H, _W), jnp.float32),
            pltpu.VMEM((_K, _H, _W), jnp.float32),
            pltpu.VMEM((_K, _H, _W), jnp.float32),
            pltpu.VMEM((_K, _H, _W), jnp.float32),
            pltpu.SemaphoreType.DMA,
            pltpu.SemaphoreType.DMA,
            pltpu.SemaphoreType.DMA,
            pltpu.SemaphoreType.DMA,
        ],
    )
    return sc(pred, y0, jnp.asarray(gxy))


# speedup vs baseline: 1.6940x; 1.6940x over previous
import numpy as np
import jax
import jax.numpy as jnp
from jax.experimental import pallas as pl

_PRIOR_BOXES = np.array([[1.3221, 1.73145], [3.19275, 4.00944], [5.05587, 8.09892],
                         [9.47112, 4.84053], [11.2364, 10.0071]], dtype=np.float32) / 13.0
_NUM_PRIOR = 5
_NUM_CLASSES = 20
_IMG_W = 416.0
_IMG_H = 416.0
_LAMBDA_OBJ = 5.0
_LAMBDA_NONOBJ = 0.5


def _coeffs(C, H, W, grid_S):
    dx = np.float32(_IMG_W / grid_S)
    dy = np.float32(_IMG_H / grid_S)
    nel = 5 + _NUM_CLASSES
    keep = np.zeros((C, 1), np.float32)
    s1 = np.zeros((C, 1), np.float32)
    s2 = np.zeros((C, 1), np.float32)
    ax = np.zeros((C, 1), np.float32)
    ay = np.zeros((C, 1), np.float32)
    for c in range(C):
        pos, i = c % nel, c // nel
        if pos == 0 or pos >= 5:
            keep[c] = 1.0
        elif pos == 1:
            s1[c], s2[c], ax[c] = dx, 1.0, 1.0
        elif pos == 2:
            s1[c], s2[c], ay[c] = dy, 1.0, 1.0
        elif pos == 3:
            s1[c], s2[c] = _PRIOR_BOXES[i, 0], _IMG_W
        else:
            s1[c], s2[c] = _PRIOR_BOXES[i, 1], _IMG_H
    cell_x = np.tile(np.arange(W, dtype=np.float32), H)
    cell_y = np.repeat(np.arange(H, dtype=np.float32), W)
    gx = (dx * cell_x).reshape(1, H * W)
    gy = (dy * cell_y).reshape(1, H * W)
    coef = np.concatenate([keep, s1, s2, ax, ay], axis=1)
    grid_vec = np.concatenate([gx, gy], axis=0)
    return coef, grid_vec


def _decode_body(p_ref, y_ref, coef_ref, g_ref, o_ref):
    keep = coef_ref[:, 0:1]
    s1 = coef_ref[:, 1:2]
    s2 = coef_ref[:, 2:3]
    ax = coef_ref[:, 3:4]
    ay = coef_ref[:, 4:5]
    gx = g_ref[0:1, :]
    gy = g_ref[1:2, :]
    nb = p_ref.shape[0]
    for b in range(nb):
        p = p_ref[b]
        y0 = y_ref[b]
        val = keep * p + jnp.trunc((s1 * p) * s2) + (ax * gx + ay * gy)
        non_obj = jnp.negative(y0 + (-1.0))
        mask = _LAMBDA_OBJ * y0 + _LAMBDA_NONOBJ * non_obj
        o_ref[b] = val * mask


def kernel(pred, y_hat):
    B, C, H, W = pred.shape
    grid_S = C
    HW = H * W
    coef, grid_vec = _coeffs(C, H, W, grid_S)

    NB = 8
    pred2 = pred.reshape(B, C, HW)
    y0 = y_hat[:, :, :, 0].reshape(B, 1, HW)

    out = pl.pallas_call(
        _decode_body,
        grid=(B // NB,),
        in_specs=[
            pl.BlockSpec((NB, C, HW), lambda b: (b, 0, 0)),
            pl.BlockSpec((NB, 1, HW), lambda b: (b, 0, 0)),
            pl.BlockSpec((C, 5), lambda b: (0, 0)),
            pl.BlockSpec((2, HW), lambda b: (0, 0)),
        ],
        out_specs=pl.BlockSpec((NB, C, HW), lambda b: (b, 0, 0)),
        out_shape=jax.ShapeDtypeStruct((B, C, HW), jnp.float32),
    )(pred2, y0, jnp.asarray(coef), jnp.asarray(grid_vec))
    return out.reshape(B, C, H, W)
